# baseline (speedup 1.0000x reference)
import math

import jax
import jax.numpy as jnp
from jax.experimental import pallas as pl

_N_POINTS = 32768
_N_UNITS = 1024
_N_DIAG = 16
_THRESH = 0.01
_TILE = 512
_N_TILES = _N_POINTS // _TILE

_C45 = math.cos(-math.pi / 4.0)
_S45 = math.sin(-math.pi / 4.0)
_SKIP_BOUND = 110.0


def _topo_kernel(xt_ref, sl_ref, mu0_ref, lmu1_ref, ls0_ref, ls1_ref, out_ref):
    c = jnp.float32(_C45)
    s = jnp.float32(_S45)
    thresh = jnp.float32(_THRESH)
    nk = jnp.float32(-1.4426950408889634)

    mu0 = mu0_ref[:]
    s0 = jnp.exp(ls0_ref[:])
    mu1 = jnp.exp(lmu1_ref[:])
    s1 = jnp.exp(ls1_ref[:])
    s0sq = s0 * s0
    s1sq = s1 * s1
    g5 = jnp.stack([nk * s0sq,
                    (-2.0 * nk) * (s0sq * mu0),
                    nk * s1sq,
                    (-2.0 * nk) * (s1sq * mu1),
                    nk * (s0sq * mu0 * mu0 + s1sq * mu1 * mu1)],
                   axis=0)
    g_hi = g5.astype(jnp.bfloat16)
    g_lo = (g5 - g_hi.astype(jnp.float32)).astype(jnp.bfloat16)
    zg = jnp.zeros((1, _N_UNITS), jnp.bfloat16)
    g_cat = jnp.concatenate([g_hi, g_lo, g_hi, zg], axis=0)

    st = sl_ref[:, 0:1]
    en = sl_ref[:, 1:2]

    out_ref[:] = jnp.zeros((_N_DIAG, _N_UNITS), jnp.float32)

    x0_all = xt_ref[:, 0, :] * c - xt_ref[:, 1, :] * s
    t_lo = jnp.min(x0_all, axis=1)[:, None]
    t_hi = jnp.max(x0_all, axis=1)[:, None]
    d = jnp.maximum(jnp.maximum(t_lo - mu0[None, :], mu0[None, :] - t_hi), 0.0)
    sd = s0[None, :] * d
    alive = jnp.min(sd * sd, axis=1) <= jnp.float32(_SKIP_BOUND)
    ti = jax.lax.broadcasted_iota(jnp.int32, (_N_TILES,), 0)
    t_min = jnp.min(jnp.where(alive, ti, _N_TILES))
    t_max = jnp.max(jnp.where(alive, ti, -1))

    def _tile_body(t, _):
        xa = xt_ref[t, 0, :]
        xb = xt_ref[t, 1, :]
        x0 = xa * c - xb * s
        x1 = xa * s + xb * c
        x1_alt = jnp.log(x1 / thresh) * thresh + thresh
        x1e = jnp.where(x0 >= thresh, x1, x1_alt)

        ones = jnp.ones_like(x0)
        f5 = jnp.stack([x0 * x0, x0, x1e * x1e, x1e, ones], axis=0)
        f_hi = f5.astype(jnp.bfloat16)
        f_lo = (f5 - f_hi.astype(jnp.float32)).astype(jnp.bfloat16)
        zf = jnp.zeros((1, _TILE), jnp.bfloat16)
        f_cat = jnp.concatenate([f_hi, f_hi, f_lo, zf], axis=0)
        e2 = jax.lax.dot_general(f_cat, g_cat, (((0,), (0,)), ((), ())),
                                 preferred_element_type=jnp.float32)
        out = jnp.exp2(e2)

        gi = t * _TILE + jax.lax.broadcasted_iota(
            jnp.int32, (_N_DIAG, _TILE), 1)
        w = ((gi >= st) & (gi < en)).astype(jnp.float32)
        contrib = jax.lax.dot(w, out, preferred_element_type=jnp.float32)
        out_ref[:] = out_ref[:] + contrib
        return _

    jax.lax.fori_loop(t_min, t_max + 1, _tile_body, None)


@jax.jit
def kernel(X_persis, diagram_slices, mu0, log_mu1, log_sigma0, log_sigma1):
    sl = diagram_slices.astype(jnp.int32)
    xt = jnp.transpose(X_persis.reshape(_N_TILES, _TILE, 2), (0, 2, 1))
    return pl.pallas_call(
        _topo_kernel,
        out_shape=jax.ShapeDtypeStruct((_N_DIAG, _N_UNITS), jnp.float32),
    )(xt, sl, mu0, log_mu1, log_sigma0, log_sigma1)
